# baseline (speedup 1.0000x reference)
import functools

import jax
import jax.numpy as jnp
from jax import lax
from jax.experimental import pallas as pl
from jax.experimental.pallas import tpu as pltpu
from jax.experimental.pallas import tpu_sc as plsc

NC, NS = 2, 16
NW = NC * NS
CH = 128
NBUF = 4


def _make_sc_gather(n_flat, d):
    n_rows = n_flat // NW
    nch = n_rows // CH

    mesh = plsc.VectorSubcoreMesh(
        core_axis_name="c", subcore_axis_name="s",
        num_cores=NC, num_subcores=NS,
    )

    @functools.partial(
        pl.kernel,
        mesh=mesh,
        compiler_params=pltpu.CompilerParams(needs_layout_passes=False),
        out_type=jax.ShapeDtypeStruct((n_flat, d), jnp.float32),
        scratch_types=[
            pltpu.VMEM((n_rows,), jnp.int32),
            *[pltpu.VMEM((CH, d), jnp.float32) for _ in range(NBUF)],
            *[pltpu.SemaphoreType.DMA for _ in range(NBUF)],
            *[pltpu.SemaphoreType.DMA for _ in range(NBUF)],
        ],
    )
    def sc_gather(idx_hbm, table_hbm, x_hbm, idx_v, *bufs_and_sems):
        bufs = bufs_and_sems[:NBUF]
        gsems = bufs_and_sems[NBUF:2 * NBUF]
        osems = bufs_and_sems[2 * NBUF:3 * NBUF]

        wid = lax.axis_index("s") * NC + lax.axis_index("c")
        base = wid * n_rows

        pltpu.sync_copy(idx_hbm.at[pl.ds(base, n_rows)], idx_v)

        def gather_src(c):
            return table_hbm.at[idx_v.at[pl.ds(c * CH, CH)]]

        def out_dst(c):
            return x_hbm.at[pl.ds(base + c * CH, CH)]

        pltpu.async_copy(gather_src(0), bufs[0], gsems[0])
        pltpu.async_copy(gather_src(1), bufs[1], gsems[1])

        def slot(c, off):
            b = (off + 2) % NBUF

            @pl.when(c >= 2)
            def _():
                pltpu.make_async_copy(bufs[b], out_dst(c - 2), osems[b]).wait()

            @pl.when(c + 2 < nch)
            def _():
                pltpu.async_copy(gather_src(c + 2), bufs[b], gsems[b])

            pltpu.make_async_copy(gather_src(c), bufs[off], gsems[off]).wait()
            pltpu.async_copy(bufs[off], out_dst(c), osems[off])

        def outer(i, _):
            for off in range(NBUF):
                slot(NBUF * i + off, off)
            return 0

        n_full = nch // NBUF
        lax.fori_loop(0, n_full, outer, 0)
        for off in range(nch - NBUF * n_full):
            slot(NBUF * n_full + off, off)

        pltpu.make_async_copy(
            bufs[(nch - 2) % NBUF], out_dst(nch - 2), osems[(nch - 2) % NBUF]
        ).wait()
        pltpu.make_async_copy(
            bufs[(nch - 1) % NBUF], out_dst(nch - 1), osems[(nch - 1) % NBUF]
        ).wait()

    return sc_gather


def _tc_ln_body(x_ref, pos_ref, g_ref, b_ref, o_ref):
    x = x_ref[...] + pos_ref[...]
    mean = jnp.mean(x, axis=1, keepdims=True)
    xc = x - mean
    var = jnp.mean(xc * xc, axis=1, keepdims=True)
    normed = xc * lax.rsqrt(var + 1e-5)
    o_ref[...] = normed * g_ref[...] + b_ref[...]


def _tc_ln(x, pos, gamma, beta, n_seq, seq_len, d):
    return pl.pallas_call(
        _tc_ln_body,
        grid=(n_seq,),
        in_specs=[
            pl.BlockSpec((seq_len, d), lambda i: (i, 0)),
            pl.BlockSpec((seq_len, d), lambda i: (0, 0)),
            pl.BlockSpec((1, d), lambda i: (0, 0)),
            pl.BlockSpec((1, d), lambda i: (0, 0)),
        ],
        out_specs=pl.BlockSpec((seq_len, d), lambda i: (i, 0)),
        out_shape=jax.ShapeDtypeStruct((n_seq * seq_len, d), jnp.float32),
        compiler_params=pltpu.CompilerParams(
            dimension_semantics=("arbitrary",),
        ),
    )(x, pos, gamma, beta)


@jax.jit
def kernel(input_ids, token_table, pos_table, ln_gamma, ln_beta):
    b, l = input_ids.shape
    _, d = token_table.shape
    ids = input_ids.reshape(b * l).astype(jnp.int32)
    x = _make_sc_gather(b * l, d)(ids, token_table)
    out = _tc_ln(
        x, pos_table[:l], ln_gamma.reshape(1, d), ln_beta.reshape(1, d),
        b, l, d,
    )
    return out.reshape(b, l, d)


# speedup vs baseline: 3.5441x; 3.5441x over previous
import functools

import jax
import jax.numpy as jnp
from jax import lax
from jax.experimental import pallas as pl
from jax.experimental.pallas import tpu as pltpu
from jax.experimental.pallas import tpu_sc as plsc

NC, NS = 2, 16
NW = NC * NS
CH = 128
NBUF = 4


def _make_sc_gather(n_flat, d):
    n_rows = n_flat // NW
    nch = n_rows // CH

    mesh = plsc.VectorSubcoreMesh(
        core_axis_name="c", subcore_axis_name="s",
        num_cores=NC, num_subcores=NS,
    )

    @functools.partial(
        pl.kernel,
        mesh=mesh,
        compiler_params=pltpu.CompilerParams(needs_layout_passes=False),
        out_type=jax.ShapeDtypeStruct((n_flat, d), jnp.float32),
        scratch_types=[
            pltpu.VMEM((n_rows,), jnp.int32),
            *[pltpu.VMEM((CH, d), jnp.float32) for _ in range(NBUF)],
            *[pltpu.SemaphoreType.DMA for _ in range(NBUF)],
            *[pltpu.SemaphoreType.DMA for _ in range(NBUF)],
        ],
    )
    def sc_gather(idx_hbm, table_hbm, x_hbm, idx_v, *bufs_and_sems):
        bufs = bufs_and_sems[:NBUF]
        gsems = bufs_and_sems[NBUF:2 * NBUF]
        osems = bufs_and_sems[2 * NBUF:3 * NBUF]

        wid = lax.axis_index("s") * NC + lax.axis_index("c")
        base = wid * n_rows

        pltpu.sync_copy(idx_hbm.at[pl.ds(base, n_rows)], idx_v)

        def gather_src(c):
            return table_hbm.at[idx_v.at[pl.ds(c * CH, CH)]]

        def out_dst(c):
            return x_hbm.at[pl.ds(base + c * CH, CH)]

        pltpu.async_copy(gather_src(0), bufs[0], gsems[0])
        pltpu.async_copy(gather_src(1), bufs[1], gsems[1])

        def slot(c, off):
            b = (off + 2) % NBUF

            @pl.when(c >= 2)
            def _():
                pltpu.make_async_copy(bufs[b], out_dst(c - 2), osems[b]).wait()

            @pl.when(c + 2 < nch)
            def _():
                pltpu.async_copy(gather_src(c + 2), bufs[b], gsems[b])

            pltpu.make_async_copy(gather_src(c), bufs[off], gsems[off]).wait()
            pltpu.async_copy(bufs[off], out_dst(c), osems[off])

        def outer(i, _):
            for off in range(NBUF):
                slot(NBUF * i + off, off)
            return 0

        n_full = nch // NBUF
        lax.fori_loop(0, n_full, outer, 0)
        for off in range(nch - NBUF * n_full):
            slot(NBUF * n_full + off, off)

        pltpu.make_async_copy(
            bufs[(nch - 2) % NBUF], out_dst(nch - 2), osems[(nch - 2) % NBUF]
        ).wait()
        pltpu.make_async_copy(
            bufs[(nch - 1) % NBUF], out_dst(nch - 1), osems[(nch - 1) % NBUF]
        ).wait()

    return sc_gather


def _tc_ln_body(x_ref, pos_ref, g_ref, b_ref, o_ref):
    x = x_ref[...] + pos_ref[...]
    mean = jnp.mean(x, axis=1, keepdims=True)
    xc = x - mean
    var = jnp.mean(xc * xc, axis=1, keepdims=True)
    normed = xc * lax.rsqrt(var + 1e-5)
    o_ref[...] = normed * g_ref[...] + b_ref[...]


SEQ_PER_BLOCK = 16


def _tc_ln(x, pos_tiled, gamma, beta, n_seq, seq_len, d):
    rows = SEQ_PER_BLOCK * seq_len
    return pl.pallas_call(
        _tc_ln_body,
        grid=(n_seq // SEQ_PER_BLOCK,),
        in_specs=[
            pl.BlockSpec((rows, d), lambda i: (i, 0)),
            pl.BlockSpec((rows, d), lambda i: (0, 0)),
            pl.BlockSpec((1, d), lambda i: (0, 0)),
            pl.BlockSpec((1, d), lambda i: (0, 0)),
        ],
        out_specs=pl.BlockSpec((rows, d), lambda i: (i, 0)),
        out_shape=jax.ShapeDtypeStruct((n_seq * seq_len, d), jnp.float32),
        compiler_params=pltpu.CompilerParams(
            dimension_semantics=("arbitrary",),
        ),
    )(x, pos_tiled, gamma, beta)


@jax.jit
def kernel(input_ids, token_table, pos_table, ln_gamma, ln_beta):
    b, l = input_ids.shape
    _, d = token_table.shape
    ids = input_ids.reshape(b * l).astype(jnp.int32)
    x = _make_sc_gather(b * l, d)(ids, token_table)
    pos_tiled = jnp.tile(pos_table[:l], (SEQ_PER_BLOCK, 1))
    out = _tc_ln(
        x, pos_tiled, ln_gamma.reshape(1, d), ln_beta.reshape(1, d),
        b, l, d,
    )
    return out.reshape(b, l, d)
